# baseline (speedup 1.0000x reference)
import jax
import jax.numpy as jnp
from jax.experimental import pallas as pl
from jax.experimental.pallas import tpu as pltpu


def _copy_body(in_ref, out_ref):
    out_ref[...] = in_ref[...]


def kernel(c_embeddings):
    n, d = c_embeddings.shape
    xt = c_embeddings.T
    bc = 32768
    grid = (pl.cdiv(n, bc),)
    out = pl.pallas_call(
        _copy_body,
        out_shape=jax.ShapeDtypeStruct((d, n), xt.dtype),
        grid=grid,
        in_specs=[pl.BlockSpec((d, bc), lambda i: (0, i))],
        out_specs=pl.BlockSpec((d, bc), lambda i: (0, i)),
    )(xt)
    return out.T


# speedup vs baseline: 1.1412x; 1.1412x over previous
import jax
import jax.numpy as jnp
from jax.experimental import pallas as pl
from jax.experimental.pallas import tpu as pltpu


def _copy_body(in_ref, out_ref):
    out_ref[...] = in_ref[...]


def kernel(c_embeddings):
    n, d = c_embeddings.shape
    xt = c_embeddings.T
    bc = 131072
    grid = (pl.cdiv(n, bc),)
    out = pl.pallas_call(
        _copy_body,
        out_shape=jax.ShapeDtypeStruct((d, n), xt.dtype),
        grid=grid,
        in_specs=[pl.BlockSpec((d, bc), lambda i: (0, i))],
        out_specs=pl.BlockSpec((d, bc), lambda i: (0, i)),
    )(xt)
    return out.T


# speedup vs baseline: 1.1560x; 1.0130x over previous
import jax
import jax.numpy as jnp
from jax.experimental import pallas as pl
from jax.experimental.pallas import tpu as pltpu


def _copy_body(in_ref, out_ref):
    out_ref[...] = in_ref[...]


def kernel(c_embeddings):
    n, d = c_embeddings.shape
    xt = c_embeddings.T
    bc = 196608
    grid = (pl.cdiv(n, bc),)
    out = pl.pallas_call(
        _copy_body,
        out_shape=jax.ShapeDtypeStruct((d, n), xt.dtype),
        grid=grid,
        in_specs=[pl.BlockSpec((d, bc), lambda i: (0, i))],
        out_specs=pl.BlockSpec((d, bc), lambda i: (0, i)),
    )(xt)
    return out.T
